# baseline (speedup 1.0000x reference)
import functools

import jax
import jax.numpy as jnp
from jax import lax
from jax.experimental import pallas as pl
from jax.experimental.pallas import tpu as pltpu
from jax.experimental.pallas import tpu_sc as plsc


@functools.lru_cache(maxsize=None)
def _build_gather(B, K, D):
    info = plsc.get_sparse_core_info()
    NC, NS = info.num_cores, info.num_subcores
    NW = NC * NS
    assert B % (8 * NW) == 0
    b_per_w = B // NW
    mesh = plsc.VectorSubcoreMesh(core_axis_name="c", subcore_axis_name="s")

    @functools.partial(
        pl.kernel,
        mesh=mesh,
        out_type=jax.ShapeDtypeStruct((B, D), jnp.float32),
        scratch_types=[
            pltpu.VMEM((b_per_w,), jnp.int32),
            pltpu.VMEM((b_per_w, 128), jnp.float32),
            pltpu.VMEM_SHARED((K, 128), jnp.float32),
            pltpu.SemaphoreType.DMA,
            pltpu.SemaphoreType.DMA,
        ],
    )
    def gather_kernel(y_hbm, table_hbm, out_hbm, idx_v, rows_v, table_sp,
                      gsem, wsem):
        sid = lax.axis_index("s")
        wid = sid * NC + lax.axis_index("c")
        base = wid * b_per_w

        @pl.when(sid == 0)
        def _stage():
            pltpu.sync_copy(table_hbm, table_sp)

        plsc.subcore_barrier()
        pltpu.sync_copy(y_hbm.at[pl.ds(base, b_per_w)], idx_v)
        pltpu.async_copy(table_sp.at[idx_v], rows_v, gsem).wait()

        def _fire(i, _):
            pltpu.make_async_copy(
                rows_v.at[i, pl.ds(0, D)], out_hbm.at[base + i], wsem
            ).start()
            return 0

        lax.fori_loop(0, b_per_w, _fire, 0)

        def _drain(i, _):
            pltpu.make_async_copy(
                rows_v.at[i, pl.ds(0, D)], out_hbm.at[base + i], wsem
            ).wait()
            return 0

        lax.fori_loop(0, b_per_w, _drain, 0)

    return gather_kernel


def kernel(y, codebook):
    (B,) = y.shape
    K, D = codebook.shape
    table = jnp.concatenate(
        [codebook, jnp.zeros((K, 128 - D), jnp.float32)], axis=1
    )
    return _build_gather(B, K, D)(y, table)
